# baseline (speedup 1.0000x reference)
import functools

import jax
import jax.numpy as jnp
from jax import lax
from jax.experimental import pallas as pl
from jax.experimental.pallas import tpu as pltpu
from jax.experimental.pallas import tpu_sc as plsc

N = 10000
D = 128
E = 320000
NC, NS = 2, 16
NW = NC * NS
B = 256
EPT = E // NS
NB = 80
PAD = NB * B - EPT
BH = 1024
NBH = (NB * B) // BH
HW = 16

NPH = 4
RBASE = (0, 2512, 5024, 7536)
RSIZE = (2512, 2512, 2512, 2464)
TRASH = 2512
NACC = 2520
RPE = 152

_MESH = plsc.VectorSubcoreMesh(
    core_axis_name="c", subcore_axis_name="s", num_cores=NC, num_subcores=NS)


@functools.partial(
    pl.kernel,
    out_type=jax.ShapeDtypeStruct((NC * N,), jnp.float32),
    mesh=_MESH,
    scratch_types=[
        pltpu.VMEM((NBH * BH,), jnp.int32),
        pltpu.VMEM((BH,), jnp.float32),
        pltpu.VMEM((RPE,), jnp.float32),
        pltpu.VMEM_SHARED((NACC,), jnp.float32),
    ],
)
def _sc_hist(d0_hbm, d1_hbm, d2_hbm, d3_hbm, ones_hbm, zeros_hbm, out_hbm,
             idx_v, ones_v, buf_v, acc_sh):
    c = lax.axis_index("c")
    s = lax.axis_index("s")
    w = c * NS + s
    pltpu.sync_copy(ones_hbm, ones_v)

    for rr, d_hbm in enumerate((d0_hbm, d1_hbm, d2_hbm, d3_hbm)):
        tailw = RSIZE[rr] - NS * RPE
        pltpu.sync_copy(d_hbm.at[w], idx_v)
        pltpu.sync_copy(zeros_hbm, buf_v)
        pltpu.sync_copy(buf_v, acc_sh.at[pl.ds(s * RPE, RPE)])

        @pl.when(s == 0)
        def _():
            pltpu.sync_copy(buf_v.at[pl.ds(0, NACC - NS * RPE)],
                            acc_sh.at[pl.ds(NS * RPE, NACC - NS * RPE)])

        plsc.subcore_barrier()

        def bat(j, carry):
            o = pl.multiple_of(j * BH, BH)
            pltpu.sync_copy(ones_v, acc_sh.at[idx_v.at[pl.ds(o, BH)]],
                            add=True)
            return carry

        lax.fori_loop(0, NBH, bat, 0)
        plsc.subcore_barrier()
        base = c * N + RBASE[rr]
        pltpu.sync_copy(acc_sh.at[pl.ds(s * RPE, RPE)], buf_v)
        pltpu.sync_copy(buf_v, out_hbm.at[pl.ds(base + s * RPE, RPE)])

        @pl.when(s == 0)
        def _():
            pltpu.sync_copy(acc_sh.at[pl.ds(NS * RPE, tailw)],
                            buf_v.at[pl.ds(0, tailw)])
            pltpu.sync_copy(buf_v.at[pl.ds(0, tailw)],
                            out_hbm.at[pl.ds(base + NS * RPE, tailw)])

        if rr < NPH - 1:
            plsc.subcore_barrier()


@functools.partial(
    pl.kernel,
    out_type=jax.ShapeDtypeStruct((NC * N, D), jnp.float32),
    mesh=_MESH,
    scratch_types=[
        pltpu.VMEM((NB * B,), jnp.int32),
        pltpu.VMEM((NB * B,), jnp.int32),
        pltpu.VMEM((2, B, D), jnp.float32),
        pltpu.VMEM_SHARED((NACC, D), jnp.float32),
        pltpu.SemaphoreType.DMA((2,)),
    ],
)
def _sc_edges(tab_hbm, src_hbm, d0_hbm, d1_hbm, d2_hbm, d3_hbm, zeros_hbm,
              out_hbm, si_v, di_v, rows_v, acc_sh, sems):
    c = lax.axis_index("c")
    s = lax.axis_index("s")
    w = c * NS + s
    pltpu.sync_copy(src_hbm.at[w], si_v)

    for rr, d_hbm in enumerate((d0_hbm, d1_hbm, d2_hbm, d3_hbm)):
        tailw = RSIZE[rr] - NS * RPE
        pltpu.sync_copy(d_hbm.at[w], di_v)
        pltpu.sync_copy(zeros_hbm, acc_sh.at[pl.ds(s * RPE, RPE)])

        @pl.when(s == 0)
        def _():
            pltpu.sync_copy(zeros_hbm.at[pl.ds(0, NACC - NS * RPE)],
                            acc_sh.at[pl.ds(NS * RPE, NACC - NS * RPE)])

        plsc.subcore_barrier()

        def grp(g, carry):
            o0 = pl.multiple_of(2 * g * B, B)
            o1 = pl.multiple_of((2 * g + 1) * B, B)
            s0 = si_v.at[pl.ds(o0, B)]
            s1 = si_v.at[pl.ds(o1, B)]
            cp0 = pltpu.async_copy(tab_hbm.at[s0], rows_v.at[0], sems.at[0])
            cp1 = pltpu.async_copy(tab_hbm.at[s1], rows_v.at[1], sems.at[1])
            cp0.wait()
            pltpu.sync_copy(rows_v.at[0], acc_sh.at[di_v.at[pl.ds(o0, B)]],
                            add=True)
            cp1.wait()
            pltpu.sync_copy(rows_v.at[1], acc_sh.at[di_v.at[pl.ds(o1, B)]],
                            add=True)
            return carry

        lax.fori_loop(0, NB // 2, grp, 0)
        plsc.subcore_barrier()
        base = c * N + RBASE[rr]
        pltpu.sync_copy(acc_sh.at[pl.ds(s * RPE, RPE)],
                        out_hbm.at[pl.ds(base + s * RPE, RPE)])

        @pl.when(s == 0)
        def _():
            pltpu.sync_copy(acc_sh.at[pl.ds(NS * RPE, tailw)],
                            out_hbm.at[pl.ds(base + NS * RPE, tailw)])

        if rr < NPH - 1:
            plsc.subcore_barrier()


_RBLK = 2000


def _mm_body(xp, xd, wpp, wtd, wpr, btd, bpr, h1, h2, x3):
    xpv = xp[...]
    h1[...] = jnp.dot(xpv, wpp[...], preferred_element_type=jnp.float32)
    h2[...] = jnp.dot(xd[...], wtd[...], preferred_element_type=jnp.float32) + btd[...]
    x3[...] = jnp.dot(xpv, wpr[...], preferred_element_type=jnp.float32) + bpr[...]


def _tc_mm(xp, xd, wpp, wtd, wpr, btd, bpr):
    g = N // _RBLK
    row = pl.BlockSpec((_RBLK, D), lambda i: (i, 0))
    full = pl.BlockSpec((D, D), lambda i: (0, 0))
    bias = pl.BlockSpec((1, D), lambda i: (0, 0))
    return pl.pallas_call(
        _mm_body,
        grid=(g,),
        in_specs=[row, row, full, full, full, bias, bias],
        out_specs=[row, row, row],
        out_shape=[jax.ShapeDtypeStruct((N, D), jnp.float32)] * 3,
    )(xp, xd, wpp, wtd, wpr, btd, bpr)


def _scale_body(h1, h2, hist, tab, dis):
    deg = hist[...] + 1.0
    d = lax.rsqrt(deg)
    tab[0] = h1[...] * d
    tab[1] = h2[...]
    dis[...] = d


def _tc_scale(h1, h2, hist_p):
    g = N // _RBLK
    row = pl.BlockSpec((_RBLK, D), lambda i: (i, 0))
    return pl.pallas_call(
        _scale_body,
        grid=(g,),
        in_specs=[row, row, pl.BlockSpec((_RBLK, 1), lambda i: (i, 0))],
        out_specs=[pl.BlockSpec((2, _RBLK, D), lambda i: (0, i, 0)),
                   pl.BlockSpec((_RBLK, 1), lambda i: (i, 0))],
        out_shape=[jax.ShapeDtypeStruct((2, N, D), jnp.float32),
                   jax.ShapeDtypeStruct((N, 1), jnp.float32)],
    )(h1, h2, hist_p)


def _final_body(acc, tab, x3, dis, hist_d, bpp, out):
    a = acc[...]
    p = a[0]
    sm = a[1]
    gx = tab[0]
    cnt = hist_d[...]
    x1 = dis[...] * (p + gx) + bpp[...]
    x2 = sm / jnp.maximum(cnt, 1.0)
    y = x1 + x2 + x3[...] + 1e-6
    m = jnp.mean(y, axis=-1, keepdims=True)
    yc = y - m
    v = jnp.mean(yc * yc, axis=-1, keepdims=True)
    out[...] = yc * lax.rsqrt(v + 1e-5)


def _tc_final(acc, tab, x3, dis, hist_d, bpp):
    g = N // _RBLK
    row = pl.BlockSpec((_RBLK, D), lambda i: (i, 0))
    return pl.pallas_call(
        _final_body,
        grid=(g,),
        in_specs=[pl.BlockSpec((2, _RBLK, D), lambda i: (0, i, 0)),
                  pl.BlockSpec((1, _RBLK, D), lambda i: (0, i, 0)),
                  row,
                  pl.BlockSpec((_RBLK, 1), lambda i: (i, 0)),
                  pl.BlockSpec((_RBLK, 1), lambda i: (i, 0)),
                  pl.BlockSpec((1, D), lambda i: (0, 0))],
        out_specs=row,
        out_shape=jax.ShapeDtypeStruct((N, D), jnp.float32),
    )(acc, tab, x3, dis, hist_d, bpp)


def kernel(x_prot, x_drug, ppi_edge_index, dti_edge_index, ddi_edge_index,
           W_pp, b_pp, W_td, b_td, W_pr, b_pr):
    del ddi_edge_index
    src_all = jnp.pad(
        jnp.concatenate([ppi_edge_index[0], dti_edge_index[0] + N])
        .reshape(NW, EPT), ((0, 0), (0, PAD)))
    dst_pad = jnp.pad(
        jnp.concatenate([ppi_edge_index[1], dti_edge_index[1]])
        .reshape(NW, EPT), ((0, 0), (0, PAD)), constant_values=-1)
    dsts = [jnp.where((dst_pad >= RBASE[rr]) & (dst_pad < RBASE[rr] + RSIZE[rr]),
                      dst_pad - RBASE[rr], TRASH)
            for rr in range(NPH)]
    ones1 = jnp.ones((BH,), jnp.float32)
    zeros1 = jnp.zeros((RPE,), jnp.float32)
    zeros128 = jnp.zeros((RPE, D), jnp.float32)

    hist = _sc_hist(*dsts, ones1, zeros1)[:, None]
    h1, h2, x3 = _tc_mm(x_prot, x_drug, W_pp, W_td, W_pr,
                        b_td.reshape(1, D), b_pr.reshape(1, D))
    tab, dis = _tc_scale(h1, h2, hist[:N])
    acc = _sc_edges(tab.reshape(NC * N, D), src_all, *dsts, zeros128)
    return _tc_final(acc.reshape(2, N, D), tab, x3, dis, hist[N:],
                     b_pp.reshape(1, D))


# speedup vs baseline: 1.7368x; 1.7368x over previous
import functools

import jax
import jax.numpy as jnp
from jax import lax
from jax.experimental import pallas as pl
from jax.experimental.pallas import tpu as pltpu
from jax.experimental.pallas import tpu_sc as plsc

N = 10000
D = 128
E = 320000
NC, NS = 2, 16
NW = NC * NS
B = 128
EPT = E // NS
NB = 160
PAD = NB * B - EPT
BH = 1024
NBH = (NB * B) // BH

NPH = 3
RBASE = (0, 3336, 6672)
RSIZE = (3336, 3336, 3328)
TRASH = 3336
NACC = 3344
RPE = 208

NACCH = N + 16
RPH = 624

_MESH = plsc.VectorSubcoreMesh(
    core_axis_name="c", subcore_axis_name="s", num_cores=NC, num_subcores=NS)


@functools.partial(
    pl.kernel,
    out_type=jax.ShapeDtypeStruct((NC * N,), jnp.float32),
    mesh=_MESH,
    scratch_types=[
        pltpu.VMEM((NBH * BH,), jnp.int32),
        pltpu.VMEM((BH,), jnp.float32),
        pltpu.VMEM((RPH,), jnp.float32),
        pltpu.VMEM_SHARED((NACCH,), jnp.float32),
    ],
)
def _sc_hist(dst_hbm, ones_hbm, zeros_hbm, out_hbm, idx_v, ones_v, buf_v,
             acc_sh):
    c = lax.axis_index("c")
    s = lax.axis_index("s")
    w = c * NS + s
    pltpu.sync_copy(ones_hbm, ones_v)
    pltpu.sync_copy(dst_hbm.at[w], idx_v)
    pltpu.sync_copy(zeros_hbm, buf_v)
    pltpu.sync_copy(buf_v, acc_sh.at[pl.ds(s * RPH, RPH)])

    @pl.when(s == 0)
    def _():
        pltpu.sync_copy(buf_v.at[pl.ds(0, NACCH - NS * RPH)],
                        acc_sh.at[pl.ds(NS * RPH, NACCH - NS * RPH)])

    plsc.subcore_barrier()

    def bat(j, carry):
        o = pl.multiple_of(j * BH, BH)
        pltpu.sync_copy(ones_v, acc_sh.at[idx_v.at[pl.ds(o, BH)]], add=True)
        return carry

    lax.fori_loop(0, NBH, bat, 0)
    plsc.subcore_barrier()
    pltpu.sync_copy(acc_sh.at[pl.ds(s * RPH, RPH)], buf_v)
    pltpu.sync_copy(buf_v, out_hbm.at[pl.ds(c * N + s * RPH, RPH)])

    @pl.when(s == 0)
    def _():
        pltpu.sync_copy(acc_sh.at[pl.ds(NS * RPH, N - NS * RPH)],
                        buf_v.at[pl.ds(0, N - NS * RPH)])
        pltpu.sync_copy(buf_v.at[pl.ds(0, N - NS * RPH)],
                        out_hbm.at[pl.ds(c * N + NS * RPH, N - NS * RPH)])


@functools.partial(
    pl.kernel,
    out_type=jax.ShapeDtypeStruct((NC * N, D), jnp.float32),
    mesh=_MESH,
    scratch_types=[
        pltpu.VMEM((NB, B), jnp.int32),
        pltpu.VMEM((NB, B), jnp.int32),
        pltpu.VMEM((B, D), jnp.float32),
        pltpu.VMEM_SHARED((NACC, D), jnp.float32),
        pltpu.SemaphoreType.DMA((2,)),
    ],
)
def _sc_edges(tab_hbm, src_hbm, d0_hbm, d1_hbm, d2_hbm, zeros_hbm,
              out_hbm, si_v, di_v, rows_v, acc_sh, sems):
    c = lax.axis_index("c")
    s = lax.axis_index("s")
    w = c * NS + s
    pltpu.sync_copy(src_hbm.at[w], si_v)

    for rr, d_hbm in enumerate((d0_hbm, d1_hbm, d2_hbm)):
        tailw = RSIZE[rr] - NS * RPE
        pltpu.sync_copy(d_hbm.at[w], di_v)
        pltpu.sync_copy(zeros_hbm, acc_sh.at[pl.ds(s * RPE, RPE)])

        @pl.when(s == 0)
        def _():
            pltpu.sync_copy(zeros_hbm.at[pl.ds(0, NACC - NS * RPE)],
                            acc_sh.at[pl.ds(NS * RPE, NACC - NS * RPE)])

        plsc.subcore_barrier()

        def bat(j, carry):
            pltpu.async_copy(tab_hbm.at[si_v.at[j]], rows_v,
                             sems.at[0]).wait()
            pltpu.sync_copy(rows_v, acc_sh.at[di_v.at[j]], add=True)
            return carry

        lax.fori_loop(0, NB, bat, 0)
        plsc.subcore_barrier()
        base = c * N + RBASE[rr]
        pltpu.sync_copy(acc_sh.at[pl.ds(s * RPE, RPE)],
                        out_hbm.at[pl.ds(base + s * RPE, RPE)])

        if tailw:
            @pl.when(s == 0)
            def _():
                pltpu.sync_copy(acc_sh.at[pl.ds(NS * RPE, tailw)],
                                out_hbm.at[pl.ds(base + NS * RPE, tailw)])

        if rr < NPH - 1:
            plsc.subcore_barrier()


_RBLK = 2000


def _mm_body(xp, xd, wpp, wtd, wpr, btd, bpr, h1, h2, x3):
    xpv = xp[...]
    h1[...] = jnp.dot(xpv, wpp[...], preferred_element_type=jnp.float32)
    h2[...] = jnp.dot(xd[...], wtd[...], preferred_element_type=jnp.float32) + btd[...]
    x3[...] = jnp.dot(xpv, wpr[...], preferred_element_type=jnp.float32) + bpr[...]


def _tc_mm(xp, xd, wpp, wtd, wpr, btd, bpr):
    g = N // _RBLK
    row = pl.BlockSpec((_RBLK, D), lambda i: (i, 0))
    full = pl.BlockSpec((D, D), lambda i: (0, 0))
    bias = pl.BlockSpec((1, D), lambda i: (0, 0))
    return pl.pallas_call(
        _mm_body,
        grid=(g,),
        in_specs=[row, row, full, full, full, bias, bias],
        out_specs=[row, row, row],
        out_shape=[jax.ShapeDtypeStruct((N, D), jnp.float32)] * 3,
    )(xp, xd, wpp, wtd, wpr, btd, bpr)


def _scale_body(h1, h2, hist, tab, dis):
    deg = hist[...] + 1.0
    d = lax.rsqrt(deg)
    tab[0] = h1[...] * d
    tab[1] = h2[...]
    dis[...] = d


def _tc_scale(h1, h2, hist_p):
    g = N // _RBLK
    row = pl.BlockSpec((_RBLK, D), lambda i: (i, 0))
    return pl.pallas_call(
        _scale_body,
        grid=(g,),
        in_specs=[row, row, pl.BlockSpec((_RBLK, 1), lambda i: (i, 0))],
        out_specs=[pl.BlockSpec((2, _RBLK, D), lambda i: (0, i, 0)),
                   pl.BlockSpec((_RBLK, 1), lambda i: (i, 0))],
        out_shape=[jax.ShapeDtypeStruct((2, N, D), jnp.float32),
                   jax.ShapeDtypeStruct((N, 1), jnp.float32)],
    )(h1, h2, hist_p)


def _final_body(acc, tab, x3, dis, hist_d, bpp, out):
    a = acc[...]
    p = a[0]
    sm = a[1]
    gx = tab[0]
    cnt = hist_d[...]
    x1 = dis[...] * (p + gx) + bpp[...]
    x2 = sm / jnp.maximum(cnt, 1.0)
    y = x1 + x2 + x3[...] + 1e-6
    m = jnp.mean(y, axis=-1, keepdims=True)
    yc = y - m
    v = jnp.mean(yc * yc, axis=-1, keepdims=True)
    out[...] = yc * lax.rsqrt(v + 1e-5)


def _tc_final(acc, tab, x3, dis, hist_d, bpp):
    g = N // _RBLK
    row = pl.BlockSpec((_RBLK, D), lambda i: (i, 0))
    return pl.pallas_call(
        _final_body,
        grid=(g,),
        in_specs=[pl.BlockSpec((2, _RBLK, D), lambda i: (0, i, 0)),
                  pl.BlockSpec((1, _RBLK, D), lambda i: (0, i, 0)),
                  row,
                  pl.BlockSpec((_RBLK, 1), lambda i: (i, 0)),
                  pl.BlockSpec((_RBLK, 1), lambda i: (i, 0)),
                  pl.BlockSpec((1, D), lambda i: (0, 0))],
        out_specs=row,
        out_shape=jax.ShapeDtypeStruct((N, D), jnp.float32),
    )(acc, tab, x3, dis, hist_d, bpp)


def kernel(x_prot, x_drug, ppi_edge_index, dti_edge_index, ddi_edge_index,
           W_pp, b_pp, W_td, b_td, W_pr, b_pr):
    del ddi_edge_index
    src_all = jnp.pad(
        jnp.concatenate([ppi_edge_index[0], dti_edge_index[0] + N])
        .reshape(NW, EPT), ((0, 0), (0, PAD)))
    dst_pad = jnp.pad(
        jnp.concatenate([ppi_edge_index[1], dti_edge_index[1]])
        .reshape(NW, EPT), ((0, 0), (0, PAD)), constant_values=-1)
    dst_h = jnp.where(dst_pad < 0, N, dst_pad)
    dsts = [jnp.where((dst_pad >= RBASE[rr]) & (dst_pad < RBASE[rr] + RSIZE[rr]),
                      dst_pad - RBASE[rr], TRASH).reshape(NW, NB, B)
            for rr in range(NPH)]
    src_all = src_all.reshape(NW, NB, B)
    ones1 = jnp.ones((BH,), jnp.float32)
    zeros1 = jnp.zeros((RPH,), jnp.float32)
    zeros128 = jnp.zeros((RPE, D), jnp.float32)

    hist = _sc_hist(dst_h, ones1, zeros1)[:, None]
    h1, h2, x3 = _tc_mm(x_prot, x_drug, W_pp, W_td, W_pr,
                        b_td.reshape(1, D), b_pr.reshape(1, D))
    tab, dis = _tc_scale(h1, h2, hist[:N])
    acc = _sc_edges(tab.reshape(NC * N, D), src_all, *dsts, zeros128)
    return _tc_final(acc.reshape(2, N, D), tab, x3, dis, hist[N:],
                     b_pp.reshape(1, D))
